# Initial kernel scaffold; baseline (speedup 1.0000x reference)
import jax
import jax.numpy as jnp
from jax.experimental import pallas as pl


def kernel(x, edge_index, edge_attr, batch, conv1_Wl, conv1_bl, conv1_Wr, pool1_W, pool1_b, conv2_Wl, conv2_bl, conv2_Wr, pool2_W, pool2_b, conv3_Wl, conv3_bl, conv3_Wr, pool3_W, pool3_b, lin1_W, lin1_b, lin2_W, lin2_b, lin3_W, lin3_b):
    raise NotImplementedError("write your pallas kernel here")


# speedup vs baseline: 1.8189x; 1.8189x over previous
import functools
import math

import jax
import jax.numpy as jnp
from jax.experimental import pallas as pl

NHID = 128
RATIO = 0.2


def _dense_body(s_ref, cnt_ref, h_ref, wl_ref, bl_ref, wr_ref, pw_ref, pb_ref,
                h1_ref, a_ref, dinv_ref, base_ref):
    s = s_ref[...]
    cnt = cnt_ref[...]
    h = h_ref[...]
    mean = s / jnp.maximum(cnt, 1.0)
    h1 = jnp.dot(mean, wl_ref[...], preferred_element_type=jnp.float32)
    h1 = h1 + bl_ref[...] + jnp.dot(h, wr_ref[...],
                                    preferred_element_type=jnp.float32)
    h1 = jnp.maximum(h1, 0.0)
    h1_ref[...] = h1
    xw = jnp.dot(h1, pw_ref[...], preferred_element_type=jnp.float32)
    deg = cnt + 1.0
    dinv = jax.lax.rsqrt(deg)
    a_ref[...] = xw * dinv
    dinv_ref[...] = dinv
    base_ref[...] = xw / deg + pb_ref[...]


def _dense_stage(s, cnt, h, Wl, bl, Wr, pW, pb):
    n = h.shape[0]
    R = 400
    grid = (n // R,)
    row = pl.BlockSpec((R, NHID), lambda i: (i, 0))
    col = pl.BlockSpec((R, 1), lambda i: (i, 0))
    full = pl.BlockSpec((NHID, NHID), lambda i: (0, 0))
    vec = pl.BlockSpec((1, NHID), lambda i: (0, 0))
    pws = pl.BlockSpec((NHID, 1), lambda i: (0, 0))
    pbs = pl.BlockSpec((1, 1), lambda i: (0, 0))
    h1, a, dinv, base = pl.pallas_call(
        _dense_body,
        grid=grid,
        in_specs=[row, col, row, full, vec, full, pws, pbs],
        out_specs=[row, col, col, col],
        out_shape=[
            jax.ShapeDtypeStruct((n, NHID), jnp.float32),
            jax.ShapeDtypeStruct((n, 1), jnp.float32),
            jax.ShapeDtypeStruct((n, 1), jnp.float32),
            jax.ShapeDtypeStruct((n, 1), jnp.float32),
        ],
    )(s, cnt.reshape(n, 1), h, Wl, bl.reshape(1, NHID), Wr, pW,
      pb.reshape(1, 1))
    return h1, a[:, 0], dinv[:, 0], base[:, 0]


def _head_body(x1_ref, x2_ref, x3_ref, w1_ref, b1_ref, w2_ref, b2_ref,
               w3_ref, b3_ref, feats_ref, out_ref):
    def readout(ref):
        v = ref[...]
        mx = jnp.max(v, axis=0, keepdims=True)
        mn = jnp.mean(v, axis=0, keepdims=True)
        return jnp.concatenate([mx, mn], axis=1)

    z = readout(x1_ref) + readout(x2_ref) + readout(x3_ref)
    z = jnp.dot(z, w1_ref[...], preferred_element_type=jnp.float32)
    z = jnp.maximum(z + b1_ref[...], 0.0)
    f = jnp.dot(z, w2_ref[...], preferred_element_type=jnp.float32)
    f = jnp.maximum(f + b2_ref[...], 0.0)
    feats_ref[...] = f
    out_ref[...] = jnp.dot(f, w3_ref[...],
                           preferred_element_type=jnp.float32) + b3_ref[...]


def _head(xk1, xk2, xk3, w1, b1, w2, b2, w3, b3):
    ncls = w3.shape[1]
    grph = w2.shape[1]
    feats, out = pl.pallas_call(
        _head_body,
        out_shape=[
            jax.ShapeDtypeStruct((1, grph), jnp.float32),
            jax.ShapeDtypeStruct((1, ncls), jnp.float32),
        ],
    )(xk1, xk2, xk3, w1, b1.reshape(1, -1), w2, b2.reshape(1, -1), w3,
      b3.reshape(1, -1))
    return feats, out


def _aggregate(h, src, dst, mask):
    n = h.shape[0]
    msg = h[src] * mask[:, None]
    s = jnp.zeros((n, NHID), h.dtype).at[dst].add(msg)
    cnt = jnp.zeros((n,), h.dtype).at[dst].add(mask)
    return s, cnt


def _level(h, src, dst, mask, Wl, bl, Wr, pW, pb, k):
    n = h.shape[0]
    s, cnt = _aggregate(h, src, dst, mask)
    h1, a, dinv, base = _dense_stage(s, cnt, h, Wl, bl, Wr, pW, pb)
    g = jnp.zeros((n,), jnp.float32).at[dst].add(a[src] * mask)
    score = jnp.tanh(dinv * g + base)
    top_scores, perm = jax.lax.top_k(score, k)
    xk = h1[perm] * top_scores[:, None]
    mapping = jnp.full((n,), -1, jnp.int32).at[perm].set(
        jnp.arange(k, dtype=jnp.int32))
    ns = mapping[src]
    nd = mapping[dst]
    valid = (ns >= 0) & (nd >= 0)
    new_mask = mask * valid.astype(h.dtype)
    ns = jnp.where(valid, ns, 0)
    nd = jnp.where(valid, nd, 0)
    return xk, ns, nd, new_mask


def kernel(x, edge_index, edge_attr, batch,
           conv1_Wl, conv1_bl, conv1_Wr, pool1_W, pool1_b,
           conv2_Wl, conv2_bl, conv2_Wr, pool2_W, pool2_b,
           conv3_Wl, conv3_bl, conv3_Wr, pool3_W, pool3_b,
           lin1_W, lin1_b, lin2_W, lin2_b, lin3_W, lin3_b):
    n = batch.shape[0]
    x = x[:n]
    src = edge_index[0]
    dst = edge_index[1]
    mask = jnp.ones((edge_attr.shape[0],), x.dtype)
    k1 = int(math.ceil(RATIO * n))
    k2 = int(math.ceil(RATIO * k1))
    k3 = int(math.ceil(RATIO * k2))
    xk1, src, dst, mask = _level(x, src, dst, mask, conv1_Wl, conv1_bl,
                                 conv1_Wr, pool1_W, pool1_b, k1)
    xk2, src, dst, mask = _level(xk1, src, dst, mask, conv2_Wl, conv2_bl,
                                 conv2_Wr, pool2_W, pool2_b, k2)
    xk3, src, dst, mask = _level(xk2, src, dst, mask, conv3_Wl, conv3_bl,
                                 conv3_Wr, pool3_W, pool3_b, k3)
    return _head(xk1, xk2, xk3, lin1_W, lin1_b, lin2_W, lin2_b,
                 lin3_W, lin3_b)
